# Initial kernel scaffold; baseline (speedup 1.0000x reference)
import jax
import jax.numpy as jnp
from jax.experimental import pallas as pl


def kernel(dp, W, b):
    raise NotImplementedError("write your pallas kernel here")


# speedup vs baseline: 11.2064x; 11.2064x over previous
import math
import functools

import numpy as np
import jax
import jax.numpy as jnp
from jax.experimental import pallas as pl

HBINS, WBINS = 6, 6
KH, KW = 5, 5
NBINS = HBINS * WBINS
F32 = jnp.float32

_CT = [math.cos(j * math.pi / HBINS) for j in range(1, HBINS)]


def _conv_matrix(W, b):
    rows, cols, widx = [], [], []
    for oh in range(HBINS):
        for ow in range(WBINS):
            for kh in range(KH):
                th = oh + kh - (KH - 1) // 2
                if th < 0 or th >= HBINS:
                    continue
                for kw in range(KW):
                    pw = (ow + kw - (KW - 1) // 2) % WBINS
                    for c in range(4):
                        rows.append(oh * WBINS + ow)
                        cols.append((th * WBINS + pw) * 4 + c)
                        widx.append(c * KH * KW + kh * KW + kw)
    rows = np.asarray(rows, np.int32)
    cols = np.asarray(cols, np.int32)
    widx = np.asarray(widx, np.int32)
    flat = jnp.zeros((NBINS * NBINS * 4,), F32)
    flat = flat.at[rows * (NBINS * 4) + cols].add(W.reshape(-1)[widx])
    return flat.reshape(NBINS, NBINS * 4).T


def _b16(v):
    u = jax.lax.bitcast_convert_type(v, jnp.uint32)
    r = (u + jnp.uint32(0x7FFF) + ((u >> 16) & jnp.uint32(1))) & jnp.uint32(0xFFFF0000)
    return jax.lax.bitcast_convert_type(r, F32)


def _geom(x, y, z):
    G = x.shape[1]
    rho_pre = jnp.sqrt((x * x + y * y) + z * z)
    rmax = jnp.max(rho_pre, axis=1, keepdims=True)
    iot = jax.lax.broadcasted_iota(jnp.int32, rho_pre.shape, 1)
    cand = jnp.where(rho_pre >= rmax, iot, G)
    fidx = jnp.min(cand, axis=1, keepdims=True)
    selm = (iot == fidx).astype(F32)
    dix = jnp.sum(x * selm, axis=1, keepdims=True)
    diy = jnp.sum(y * selm, axis=1, keepdims=True)
    diz = jnp.sum(z * selm, axis=1, keepdims=True)

    den1 = jnp.sqrt((dix * dix + diy * diy) + diz * diz) + 1e-12
    ax, ay, az = dix / den1, diy / den1, diz / den1
    den2 = jnp.sqrt((ax * ax + ay * ay) + az * az) + 1e-8
    ax, ay, az = ax / den2, ay / den2, az / den2

    s = jnp.sqrt(ax * ax + ay * ay)
    c = az
    near_zero = (s < 1e-8) & (c > 0)
    near_pi = (s < 1e-8) & (c < 0)
    use_e0 = jnp.abs(ax) < 0.9
    vax = jnp.where(use_e0, 0.0, -az)
    vay = jnp.where(use_e0, az, 0.0)
    vaz = jnp.where(use_e0, -ay, ax)
    dena = jnp.sqrt((vax * vax + vay * vay) + vaz * vaz) + 1e-8
    vax, vay, vaz = vax / dena, vay / dena, vaz / dena
    vx = jnp.where(near_pi, vax, ay)
    vy = jnp.where(near_pi, vay, -ax)
    vz = jnp.where(near_pi, vaz, 0.0)
    denk = jnp.sqrt((vx * vx + vy * vy) + vz * vz) + 1e-8
    kx, ky, kz = vx / denk, vy / denk, vz / denk

    zero = jnp.zeros_like(kx)
    K = [[zero, -kz, ky], [kz, zero, -kx], [-ky, kx, zero]]
    Kb = [[_b16(e) for e in row] for row in K]
    K2 = [[(Kb[i][0] * Kb[0][j] + Kb[i][1] * Kb[1][j]) + Kb[i][2] * Kb[2][j]
           for j in range(3)] for i in range(3)]
    t1 = 1.0 - c
    Rm = [[(1.0 if i == j else 0.0) + K[i][j] * s + K2[i][j] * t1
           for j in range(3)] for i in range(3)]
    Rm = [[jnp.where(near_zero, 1.0 if i == j else 0.0, Rm[i][j])
           for j in range(3)] for i in range(3)]

    xb, yb, zb = _b16(x), _b16(y), _b16(z)
    Rb = [[_b16(Rm[i][j]) for j in range(3)] for i in range(3)]
    xr = (xb * Rb[0][0] + yb * Rb[0][1]) + zb * Rb[0][2]
    yr = (xb * Rb[1][0] + yb * Rb[1][1]) + zb * Rb[1][2]
    zr = (xb * Rb[2][0] + yb * Rb[2][1]) + zb * Rb[2][2]

    rho = jnp.maximum(jnp.sqrt((xr * xr + yr * yr) + zr * zr), 1e-12)
    ct = zr / rho
    it_ = ((ct < _CT[0]).astype(jnp.int32) + (ct < _CT[1]).astype(jnp.int32)
           + (ct < _CT[2]).astype(jnp.int32) + (ct < _CT[3]).astype(jnp.int32)
           + (ct < _CT[4]).astype(jnp.int32))

    half = 0.5 * jnp.sqrt(xr * xr + yr * yr)
    ipos = (xr < half).astype(jnp.int32) + (xr < -half).astype(jnp.int32)
    ineg = 3 + (xr >= -half).astype(jnp.int32) + (xr >= half).astype(jnp.int32)
    ip_ = jnp.where(yr > 0, ipos,
                    jnp.where(yr < 0, ineg,
                              jnp.where(xr < 0, 2, 0)))

    rho_max = jnp.max(rho, axis=1, keepdims=True)
    hf = jnp.maximum(0.5 - rho / (2.0 * (rho_max + 1e-12)), 0.0) + 0.5
    return xr, yr, zr, hf, it_ * WBINS + ip_


def _tc_body(dp_ref, mt_ref, b_ref, out_ref):
    d = dp_ref[0]
    x, y, z = d[0], d[1], d[2]
    xr, yr, zr, hf, bins = _geom(x, y, z)

    feats = (xr, yr, zr, hf)
    cols = []
    for j in range(NBINS):
        mj = (bins == j).astype(F32)
        for f in feats:
            cols.append(jnp.sum(f * mj, axis=1, keepdims=True))
    hist = jnp.concatenate(cols, axis=1)
    yconv = jnp.dot(hist, mt_ref[...], preferred_element_type=F32) + b_ref[0, 0]

    acc = jnp.zeros(x.shape, F32)
    for j in range(NBINS):
        acc = acc + (bins == j).astype(F32) * yconv[:, j:j + 1]
    out_ref[0, 0] = acc


@jax.jit
def kernel(dp, W, b):
    B, _, N, G = dp.shape
    TN = 256
    mt = _conv_matrix(W, b)
    grid = (B, N // TN)
    out = pl.pallas_call(
        _tc_body,
        grid=grid,
        in_specs=[
            pl.BlockSpec((1, 3, TN, G), lambda i, j: (i, 0, j, 0)),
            pl.BlockSpec((4 * NBINS, NBINS), lambda i, j: (0, 0)),
            pl.BlockSpec((1, 1), lambda i, j: (0, 0)),
        ],
        out_specs=pl.BlockSpec((1, 1, TN, G), lambda i, j: (i, 0, j, 0)),
        out_shape=jax.ShapeDtypeStruct((B, 1, N, G), F32),
    )(dp, mt, b.reshape(1, 1))
    return out


# speedup vs baseline: 13.3958x; 1.1954x over previous
import math
import functools

import numpy as np
import jax
import jax.numpy as jnp
from jax import lax
from jax.experimental import pallas as pl
from jax.experimental.pallas import tpu as pltpu, tpu_sc as plsc

HBINS, WBINS = 6, 6
KH, KW = 5, 5
NBINS = HBINS * WBINS
F32 = jnp.float32

_CT = [math.cos(j * math.pi / HBINS) for j in range(1, HBINS)]


def _conv_matrix(W, b):
    rows, cols, widx = [], [], []
    for oh in range(HBINS):
        for ow in range(WBINS):
            for kh in range(KH):
                th = oh + kh - (KH - 1) // 2
                if th < 0 or th >= HBINS:
                    continue
                for kw in range(KW):
                    pw = (ow + kw - (KW - 1) // 2) % WBINS
                    for c in range(4):
                        rows.append(oh * WBINS + ow)
                        cols.append((th * WBINS + pw) * 4 + c)
                        widx.append(c * KH * KW + kh * KW + kw)
    rows = np.asarray(rows, np.int32)
    cols = np.asarray(cols, np.int32)
    widx = np.asarray(widx, np.int32)
    flat = jnp.zeros((NBINS * NBINS * 4,), F32)
    flat = flat.at[rows * (NBINS * 4) + cols].add(W.reshape(-1)[widx])
    return flat.reshape(NBINS, NBINS * 4).T


def _b16(v):
    u = jax.lax.bitcast_convert_type(v, jnp.uint32)
    r = (u + jnp.uint32(0x7FFF) + ((u >> 16) & jnp.uint32(1))) & jnp.uint32(0xFFFF0000)
    return jax.lax.bitcast_convert_type(r, F32)


def _geom(x, y, z):
    G = x.shape[1]
    rho_pre = jnp.sqrt((x * x + y * y) + z * z)
    rmax = jnp.max(rho_pre, axis=1, keepdims=True)
    iot = jax.lax.broadcasted_iota(jnp.int32, rho_pre.shape, 1)
    cand = jnp.where(rho_pre >= rmax, iot, G)
    fidx = jnp.min(cand, axis=1, keepdims=True)
    selm = (iot == fidx).astype(F32)
    dix = jnp.sum(x * selm, axis=1, keepdims=True)
    diy = jnp.sum(y * selm, axis=1, keepdims=True)
    diz = jnp.sum(z * selm, axis=1, keepdims=True)

    den1 = jnp.sqrt((dix * dix + diy * diy) + diz * diz) + 1e-12
    ax, ay, az = dix / den1, diy / den1, diz / den1
    den2 = jnp.sqrt((ax * ax + ay * ay) + az * az) + 1e-8
    ax, ay, az = ax / den2, ay / den2, az / den2

    s = jnp.sqrt(ax * ax + ay * ay)
    c = az
    near_zero = (s < 1e-8) & (c > 0)
    near_pi = (s < 1e-8) & (c < 0)
    use_e0 = jnp.abs(ax) < 0.9
    vax = jnp.where(use_e0, 0.0, -az)
    vay = jnp.where(use_e0, az, 0.0)
    vaz = jnp.where(use_e0, -ay, ax)
    dena = jnp.sqrt((vax * vax + vay * vay) + vaz * vaz) + 1e-8
    vax, vay, vaz = vax / dena, vay / dena, vaz / dena
    vx = jnp.where(near_pi, vax, ay)
    vy = jnp.where(near_pi, vay, -ax)
    vz = jnp.where(near_pi, vaz, 0.0)
    denk = jnp.sqrt((vx * vx + vy * vy) + vz * vz) + 1e-8
    kx, ky, kz = vx / denk, vy / denk, vz / denk

    zero = jnp.zeros_like(kx)
    K = [[zero, -kz, ky], [kz, zero, -kx], [-ky, kx, zero]]
    Kb = [[_b16(e) for e in row] for row in K]
    K2 = [[(Kb[i][0] * Kb[0][j] + Kb[i][1] * Kb[1][j]) + Kb[i][2] * Kb[2][j]
           for j in range(3)] for i in range(3)]
    t1 = 1.0 - c
    Rm = [[(1.0 if i == j else 0.0) + K[i][j] * s + K2[i][j] * t1
           for j in range(3)] for i in range(3)]
    Rm = [[jnp.where(near_zero, 1.0 if i == j else 0.0, Rm[i][j])
           for j in range(3)] for i in range(3)]

    xb, yb, zb = _b16(x), _b16(y), _b16(z)
    Rb = [[_b16(Rm[i][j]) for j in range(3)] for i in range(3)]
    xr = (xb * Rb[0][0] + yb * Rb[0][1]) + zb * Rb[0][2]
    yr = (xb * Rb[1][0] + yb * Rb[1][1]) + zb * Rb[1][2]
    zr = (xb * Rb[2][0] + yb * Rb[2][1]) + zb * Rb[2][2]

    rho = jnp.maximum(jnp.sqrt((xr * xr + yr * yr) + zr * zr), 1e-12)
    ct = zr / rho
    it_ = ((ct < _CT[0]).astype(jnp.int32) + (ct < _CT[1]).astype(jnp.int32)
           + (ct < _CT[2]).astype(jnp.int32) + (ct < _CT[3]).astype(jnp.int32)
           + (ct < _CT[4]).astype(jnp.int32))

    half = 0.5 * jnp.sqrt(xr * xr + yr * yr)
    ipos = (xr < half).astype(jnp.int32) + (xr < -half).astype(jnp.int32)
    ineg = 3 + (xr >= -half).astype(jnp.int32) + (xr >= half).astype(jnp.int32)
    ip_ = jnp.where(yr > 0, ipos,
                    jnp.where(yr < 0, ineg,
                              jnp.where(xr < 0, 2, 0)))

    rho_max = jnp.max(rho, axis=1, keepdims=True)
    hf = jnp.maximum(0.5 - rho / (2.0 * (rho_max + 1e-12)), 0.0) + 0.5
    return xr, yr, zr, hf, it_ * WBINS + ip_


def _feat_body(dp_ref, vals_ref, sidx_ref, gidx_ref):
    d = dp_ref[0]
    x, y, z = d[0], d[1], d[2]
    xr, yr, zr, hf, bins = _geom(x, y, z)
    vals_ref[...] = jnp.concatenate([xr, yr, zr, hf], axis=1)
    rowi = jax.lax.broadcasted_iota(jnp.int32, bins.shape, 0)
    sb = (rowi * NBINS + bins) * 4
    sidx_ref[...] = jnp.concatenate([sb, sb + 1, sb + 2, sb + 3], axis=1)
    j = pl.program_id(1)
    gidx_ref[...] = ((j % 2) * 512 + rowi) * NBINS + bins


def _conv_body(hist_ref, mt_ref, b_ref, y_ref):
    y_ref[...] = (jnp.dot(hist_ref[...], mt_ref[...], preferred_element_type=F32)
                  + b_ref[0, 0])


def _make_sc_kernels(BN, G):
    mesh = plsc.VectorSubcoreMesh(core_axis_name="c", subcore_axis_name="s")
    A_TILE = BN // 32
    CHUNK = 512
    SLAB = 64
    VPC = 4 * G
    HROW = NBINS * 4

    @functools.partial(
        pl.kernel, mesh=mesh,
        compiler_params=pltpu.CompilerParams(needs_layout_passes=False),
        out_type=jax.ShapeDtypeStruct((BN * HROW,), F32),
        scratch_types=[
            pltpu.VMEM((SLAB * VPC,), F32),
            pltpu.VMEM((SLAB * VPC,), jnp.int32),
            pltpu.VMEM((CHUNK * HROW,), F32),
        ],
    )
    def scatter_k(vals_hbm, sidx_hbm, hist_hbm, vals_v, sidx_v, hist_v):
        wid = lax.axis_index("s") * 2 + lax.axis_index("c")
        iota16 = lax.broadcasted_iota(jnp.int32, (16,), 0)
        iota_pos = iota16 * VPC
        for chunk in range(A_TILE // CHUNK):
            abase = wid * A_TILE + chunk * CHUNK

            def zbody(k, _):
                hist_v[pl.ds(k * 16, 16)] = jnp.zeros((16,), F32)
                return 0
            lax.fori_loop(0, CHUNK * HROW // 16, zbody, 0)

            for slab in range(CHUNK // SLAB):
                vbase = (abase + slab * SLAB) * VPC
                pltpu.sync_copy(vals_hbm.at[pl.ds(vbase, SLAB * VPC)], vals_v)
                pltpu.sync_copy(sidx_hbm.at[pl.ds(vbase, SLAB * VPC)], sidx_v)

                def sbody(i, _):
                    pos = iota_pos + ((i >> 8) << 12) + (i & 255)
                    v = plsc.load_gather(vals_v, [pos])
                    si = plsc.load_gather(sidx_v, [pos])
                    plsc.addupdate_scatter(hist_v, [si], v)
                    return 0
                lax.fori_loop(0, (SLAB // 16) * VPC, sbody, 0)
            pltpu.sync_copy(hist_v, hist_hbm.at[pl.ds(abase * HROW, CHUNK * HROW)])

    P_TILE = A_TILE * G
    PSLAB = 8192

    @functools.partial(
        pl.kernel, mesh=mesh,
        compiler_params=pltpu.CompilerParams(needs_layout_passes=False),
        out_type=jax.ShapeDtypeStruct((BN * G,), F32),
        scratch_types=[
            pltpu.VMEM((A_TILE * NBINS,), F32),
            pltpu.VMEM((PSLAB,), jnp.int32),
            pltpu.VMEM((PSLAB,), F32),
        ],
    )
    def gather_k(y_hbm, gidx_hbm, out_hbm, y_v, gi_v, out_v):
        wid = lax.axis_index("s") * 2 + lax.axis_index("c")
        pltpu.sync_copy(y_hbm.at[pl.ds(wid * A_TILE * NBINS, A_TILE * NBINS)], y_v)
        for slab in range(P_TILE // PSLAB):
            pbase = wid * P_TILE + slab * PSLAB
            pltpu.sync_copy(gidx_hbm.at[pl.ds(pbase, PSLAB)], gi_v)

            def gbody(i, _):
                iv = gi_v[pl.ds(i * 16, 16)]
                out_v[pl.ds(i * 16, 16)] = plsc.load_gather(y_v, [iv])
                return 0
            lax.fori_loop(0, PSLAB // 16, gbody, 0)
            pltpu.sync_copy(out_v, out_hbm.at[pl.ds(pbase, PSLAB)])

    return scatter_k, gather_k


@jax.jit
def kernel(dp, W, b):
    B, _, N, G = dp.shape
    BN = B * N
    TN = 512
    mt = _conv_matrix(W, b)

    vals, sidx, gidx = pl.pallas_call(
        _feat_body,
        grid=(B, N // TN),
        in_specs=[pl.BlockSpec((1, 3, TN, G), lambda i, j: (i, 0, j, 0))],
        out_specs=[
            pl.BlockSpec((TN, 4 * G), lambda i, j: (i * (N // TN) + j, 0)),
            pl.BlockSpec((TN, 4 * G), lambda i, j: (i * (N // TN) + j, 0)),
            pl.BlockSpec((TN, G), lambda i, j: (i * (N // TN) + j, 0)),
        ],
        out_shape=[
            jax.ShapeDtypeStruct((BN, 4 * G), F32),
            jax.ShapeDtypeStruct((BN, 4 * G), jnp.int32),
            jax.ShapeDtypeStruct((BN, G), jnp.int32),
        ],
    )(dp)

    scatter_k, gather_k = _make_sc_kernels(BN, G)
    hist = scatter_k(vals.reshape(-1), sidx.reshape(-1))

    TM = 2048
    y = pl.pallas_call(
        _conv_body,
        grid=(BN // TM,),
        in_specs=[
            pl.BlockSpec((TM, 4 * NBINS), lambda i: (i, 0)),
            pl.BlockSpec((4 * NBINS, NBINS), lambda i: (0, 0)),
            pl.BlockSpec((1, 1), lambda i: (0, 0)),
        ],
        out_specs=pl.BlockSpec((TM, NBINS), lambda i: (i, 0)),
        out_shape=jax.ShapeDtypeStruct((BN, NBINS), F32),
    )(hist.reshape(BN, 4 * NBINS), mt, b.reshape(1, 1))

    out = gather_k(y.reshape(-1), gidx.reshape(-1))
    return out.reshape(B, N, G)[:, None]


# speedup vs baseline: 21.4066x; 1.5980x over previous
import math
import functools

import numpy as np
import jax
import jax.numpy as jnp
from jax import lax
from jax.experimental import pallas as pl
from jax.experimental.pallas import tpu as pltpu, tpu_sc as plsc

HBINS, WBINS = 6, 6
KH, KW = 5, 5
NBINS = HBINS * WBINS
HPAD = NBINS * 4 + 1
YPAD = NBINS + 1
F32 = jnp.float32

_CT = [math.cos(j * math.pi / HBINS) for j in range(1, HBINS)]


def _conv_matrix(W, b):
    rows, cols, widx = [], [], []
    for oh in range(HBINS):
        for ow in range(WBINS):
            for kh in range(KH):
                th = oh + kh - (KH - 1) // 2
                if th < 0 or th >= HBINS:
                    continue
                for kw in range(KW):
                    pw = (ow + kw - (KW - 1) // 2) % WBINS
                    for c in range(4):
                        rows.append(oh * WBINS + ow)
                        cols.append((th * WBINS + pw) * 4 + c)
                        widx.append(c * KH * KW + kh * KW + kw)
    sel = np.zeros((HPAD * YPAD, KH * KW * 4), np.float32)
    for r, col, w in zip(rows, cols, widx):
        sel[col * YPAD + r, w] += 1.0
    return jnp.matmul(jnp.asarray(sel), W.reshape(-1)).reshape(HPAD, YPAD)


def _b16(v):
    u = jax.lax.bitcast_convert_type(v, jnp.uint32)
    r = (u + jnp.uint32(0x7FFF) + ((u >> 16) & jnp.uint32(1))) & jnp.uint32(0xFFFF0000)
    return jax.lax.bitcast_convert_type(r, F32)


def _geom(x, y, z):
    G = x.shape[1]
    rho_pre = jnp.sqrt((x * x + y * y) + z * z)
    rmax = jnp.max(rho_pre, axis=1, keepdims=True)
    iot = jax.lax.broadcasted_iota(jnp.int32, rho_pre.shape, 1)
    cand = jnp.where(rho_pre >= rmax, iot, G)
    fidx = jnp.min(cand, axis=1, keepdims=True)
    selm = (iot == fidx).astype(F32)
    dix = jnp.sum(x * selm, axis=1, keepdims=True)
    diy = jnp.sum(y * selm, axis=1, keepdims=True)
    diz = jnp.sum(z * selm, axis=1, keepdims=True)

    den1 = jnp.sqrt((dix * dix + diy * diy) + diz * diz) + 1e-12
    ax, ay, az = dix / den1, diy / den1, diz / den1
    den2 = jnp.sqrt((ax * ax + ay * ay) + az * az) + 1e-8
    ax, ay, az = ax / den2, ay / den2, az / den2

    s = jnp.sqrt(ax * ax + ay * ay)
    c = az
    near_zero = (s < 1e-8) & (c > 0)
    near_pi = (s < 1e-8) & (c < 0)
    use_e0 = jnp.abs(ax) < 0.9
    vax = jnp.where(use_e0, 0.0, -az)
    vay = jnp.where(use_e0, az, 0.0)
    vaz = jnp.where(use_e0, -ay, ax)
    dena = jnp.sqrt((vax * vax + vay * vay) + vaz * vaz) + 1e-8
    vax, vay, vaz = vax / dena, vay / dena, vaz / dena
    vx = jnp.where(near_pi, vax, ay)
    vy = jnp.where(near_pi, vay, -ax)
    vz = jnp.where(near_pi, vaz, 0.0)
    denk = jnp.sqrt((vx * vx + vy * vy) + vz * vz) + 1e-8
    kx, ky, kz = vx / denk, vy / denk, vz / denk

    zero = jnp.zeros_like(kx)
    K = [[zero, -kz, ky], [kz, zero, -kx], [-ky, kx, zero]]
    Kb = [[_b16(e) for e in row] for row in K]
    K2 = [[(Kb[i][0] * Kb[0][j] + Kb[i][1] * Kb[1][j]) + Kb[i][2] * Kb[2][j]
           for j in range(3)] for i in range(3)]
    t1 = 1.0 - c
    Rm = [[(1.0 if i == j else 0.0) + K[i][j] * s + K2[i][j] * t1
           for j in range(3)] for i in range(3)]
    Rm = [[jnp.where(near_zero, 1.0 if i == j else 0.0, Rm[i][j])
           for j in range(3)] for i in range(3)]

    xb, yb, zb = _b16(x), _b16(y), _b16(z)
    Rb = [[_b16(Rm[i][j]) for j in range(3)] for i in range(3)]
    xr = (xb * Rb[0][0] + yb * Rb[0][1]) + zb * Rb[0][2]
    yr = (xb * Rb[1][0] + yb * Rb[1][1]) + zb * Rb[1][2]
    zr = (xb * Rb[2][0] + yb * Rb[2][1]) + zb * Rb[2][2]

    rho = jnp.maximum(jnp.sqrt((xr * xr + yr * yr) + zr * zr), 1e-12)
    ct = zr / rho
    it_ = ((ct < _CT[0]).astype(jnp.int32) + (ct < _CT[1]).astype(jnp.int32)
           + (ct < _CT[2]).astype(jnp.int32) + (ct < _CT[3]).astype(jnp.int32)
           + (ct < _CT[4]).astype(jnp.int32))

    half = 0.5 * jnp.sqrt(xr * xr + yr * yr)
    ipos = (xr < half).astype(jnp.int32) + (xr < -half).astype(jnp.int32)
    ineg = 3 + (xr >= -half).astype(jnp.int32) + (xr >= half).astype(jnp.int32)
    ip_ = jnp.where(yr > 0, ipos,
                    jnp.where(yr < 0, ineg,
                              jnp.where(xr < 0, 2, 0)))

    rho_max = jnp.max(rho, axis=1, keepdims=True)
    hf = jnp.maximum(0.5 - rho / (2.0 * (rho_max + 1e-12)), 0.0) + 0.5
    return xr, yr, zr, hf, it_ * WBINS + ip_


def _feat_body(dp_ref, vals_ref, sidx_ref, gidx_ref):
    d = dp_ref[0]
    x, y, z = d[0], d[1], d[2]
    xr, yr, zr, hf, bins = _geom(x, y, z)
    vals_ref[...] = jnp.concatenate(
        [xr.T, yr.T, zr.T, hf.T], axis=0)
    colI = jax.lax.broadcasted_iota(jnp.int32, (x.shape[1], x.shape[0]), 1)
    base_t = bins.T * 4 + colI * HPAD
    sidx_ref[...] = jnp.concatenate(
        [base_t, base_t + 1, base_t + 2, base_t + 3], axis=0)
    rowi = jax.lax.broadcasted_iota(jnp.int32, bins.shape, 0)
    j = pl.program_id(1)
    gidx_ref[...] = ((j % 2) * 512 + rowi) * YPAD + bins


def _conv_body(hist_ref, mt_ref, b_ref, y_ref):
    y_ref[...] = (jnp.dot(hist_ref[...], mt_ref[...], preferred_element_type=F32)
                  + b_ref[0, 0])


def _make_sc_kernels(BN, G):
    mesh = plsc.VectorSubcoreMesh(core_axis_name="c", subcore_axis_name="s")
    A_TILE = BN // 32
    CHUNK = 512
    VPC = 4 * G
    SLABR = 32
    SLABW = SLABR * CHUNK

    @functools.partial(
        pl.kernel, mesh=mesh,
        compiler_params=pltpu.CompilerParams(needs_layout_passes=False),
        out_type=jax.ShapeDtypeStruct((BN * HPAD,), F32),
        scratch_types=[
            pltpu.VMEM((SLABW,), F32),
            pltpu.VMEM((SLABW,), jnp.int32),
            pltpu.VMEM((CHUNK * HPAD,), F32),
        ],
    )
    def scatter_k(vals_hbm, sidx_hbm, hist_hbm, vals_v, sidx_v, hist_v):
        wid = lax.axis_index("s") * 2 + lax.axis_index("c")
        zv = jnp.zeros((16,), F32)
        for chunk in range(A_TILE // CHUNK):
            cglob = wid * (A_TILE // CHUNK) + chunk
            abase = cglob * CHUNK

            def zbody(k, _):
                b0 = k * 128
                for u in range(8):
                    hist_v[pl.ds(b0 + u * 16, 16)] = zv
                return 0
            lax.fori_loop(0, CHUNK * HPAD // 128, zbody, 0)

            for slab in range(VPC // SLABR):
                vbase = cglob * (CHUNK * VPC) + slab * SLABW
                pltpu.sync_copy(vals_hbm.at[pl.ds(vbase, SLABW)], vals_v)
                pltpu.sync_copy(sidx_hbm.at[pl.ds(vbase, SLABW)], sidx_v)

                def sbody(i, _):
                    b0 = i * 128
                    for u in range(8):
                        off = pl.ds(b0 + u * 16, 16)
                        plsc.addupdate_scatter(hist_v, [sidx_v[off]], vals_v[off])
                    return 0
                lax.fori_loop(0, SLABW // 128, sbody, 0)
            pltpu.sync_copy(hist_v, hist_hbm.at[pl.ds(abase * HPAD, CHUNK * HPAD)])

    P_TILE = A_TILE * G
    PSLAB = 8192

    @functools.partial(
        pl.kernel, mesh=mesh,
        compiler_params=pltpu.CompilerParams(needs_layout_passes=False),
        out_type=jax.ShapeDtypeStruct((BN * G,), F32),
        scratch_types=[
            pltpu.VMEM((A_TILE * YPAD,), F32),
            pltpu.VMEM((PSLAB,), jnp.int32),
            pltpu.VMEM((PSLAB,), F32),
        ],
    )
    def gather_k(y_hbm, gidx_hbm, out_hbm, y_v, gi_v, out_v):
        wid = lax.axis_index("s") * 2 + lax.axis_index("c")
        pltpu.sync_copy(y_hbm.at[pl.ds(wid * A_TILE * YPAD, A_TILE * YPAD)], y_v)
        for slab in range(P_TILE // PSLAB):
            pbase = wid * P_TILE + slab * PSLAB
            pltpu.sync_copy(gidx_hbm.at[pl.ds(pbase, PSLAB)], gi_v)

            def gbody(i, _):
                b0 = i * 128
                for u in range(8):
                    off = pl.ds(b0 + u * 16, 16)
                    out_v[off] = plsc.load_gather(y_v, [gi_v[off]])
                return 0
            lax.fori_loop(0, PSLAB // 128, gbody, 0)
            pltpu.sync_copy(out_v, out_hbm.at[pl.ds(pbase, PSLAB)])

    return scatter_k, gather_k


@jax.jit
def kernel(dp, W, b):
    B, _, N, G = dp.shape
    BN = B * N
    TN = 512
    mt = _conv_matrix(W, b)

    nblk = (B * N) // TN
    vals, sidx, gidx = pl.pallas_call(
        _feat_body,
        grid=(B, N // TN),
        in_specs=[pl.BlockSpec((1, 3, TN, G), lambda i, j: (i, 0, j, 0))],
        out_specs=[
            pl.BlockSpec((4 * G, TN), lambda i, j: (i * (N // TN) + j, 0)),
            pl.BlockSpec((4 * G, TN), lambda i, j: (i * (N // TN) + j, 0)),
            pl.BlockSpec((TN, G), lambda i, j: (i * (N // TN) + j, 0)),
        ],
        out_shape=[
            jax.ShapeDtypeStruct((nblk * 4 * G, TN), F32),
            jax.ShapeDtypeStruct((nblk * 4 * G, TN), jnp.int32),
            jax.ShapeDtypeStruct((BN, G), jnp.int32),
        ],
    )(dp)

    scatter_k, gather_k = _make_sc_kernels(BN, G)
    hist = scatter_k(vals.reshape(-1), sidx.reshape(-1))

    TM = 2048
    y = pl.pallas_call(
        _conv_body,
        grid=(BN // TM,),
        in_specs=[
            pl.BlockSpec((TM, HPAD), lambda i: (i, 0)),
            pl.BlockSpec((HPAD, YPAD), lambda i: (0, 0)),
            pl.BlockSpec((1, 1), lambda i: (0, 0)),
        ],
        out_specs=pl.BlockSpec((TM, YPAD), lambda i: (i, 0)),
        out_shape=jax.ShapeDtypeStruct((BN, YPAD), F32),
    )(hist.reshape(BN, HPAD), mt, b.reshape(1, 1))

    out = gather_k(y.reshape(-1), gidx.reshape(-1))
    return out.reshape(B, N, G)[:, None]


# speedup vs baseline: 23.6675x; 1.1056x over previous
import math
import functools

import numpy as np
import jax
import jax.numpy as jnp
from jax import lax
from jax.experimental import pallas as pl
from jax.experimental.pallas import tpu as pltpu, tpu_sc as plsc

HBINS, WBINS = 6, 6
KH, KW = 5, 5
NBINS = HBINS * WBINS
HPAD = NBINS * 4 + 1
YPAD = NBINS + 1
F32 = jnp.float32

_CT = [math.cos(j * math.pi / HBINS) for j in range(1, HBINS)]


def _conv_matrix(W, b):
    rows, cols, widx = [], [], []
    for oh in range(HBINS):
        for ow in range(WBINS):
            for kh in range(KH):
                th = oh + kh - (KH - 1) // 2
                if th < 0 or th >= HBINS:
                    continue
                for kw in range(KW):
                    pw = (ow + kw - (KW - 1) // 2) % WBINS
                    for c in range(4):
                        rows.append(oh * WBINS + ow)
                        cols.append((th * WBINS + pw) * 4 + c)
                        widx.append(c * KH * KW + kh * KW + kw)
    sel = np.zeros((HPAD * YPAD, KH * KW * 4), np.float32)
    for r, col, w in zip(rows, cols, widx):
        sel[col * YPAD + r, w] += 1.0
    return jnp.matmul(jnp.asarray(sel), W.reshape(-1)).reshape(HPAD, YPAD)


def _b16(v):
    u = jax.lax.bitcast_convert_type(v, jnp.uint32)
    r = (u + jnp.uint32(0x7FFF) + ((u >> 16) & jnp.uint32(1))) & jnp.uint32(0xFFFF0000)
    return jax.lax.bitcast_convert_type(r, F32)


def _geom(x, y, z):
    G = x.shape[1]
    rho_pre = jnp.sqrt((x * x + y * y) + z * z)
    rmax = jnp.max(rho_pre, axis=1, keepdims=True)
    iot = jax.lax.broadcasted_iota(jnp.int32, rho_pre.shape, 1)
    cand = jnp.where(rho_pre >= rmax, iot, G)
    fidx = jnp.min(cand, axis=1, keepdims=True)
    selm = (iot == fidx).astype(F32)
    dix = jnp.sum(x * selm, axis=1, keepdims=True)
    diy = jnp.sum(y * selm, axis=1, keepdims=True)
    diz = jnp.sum(z * selm, axis=1, keepdims=True)

    TN = x.shape[0]
    cshape = (TN // 128, 128)
    dix, diy, diz = (v.reshape(cshape) for v in (dix, diy, diz))

    den1 = jnp.sqrt((dix * dix + diy * diy) + diz * diz) + 1e-12
    ax, ay, az = dix / den1, diy / den1, diz / den1
    den2 = jnp.sqrt((ax * ax + ay * ay) + az * az) + 1e-8
    ax, ay, az = ax / den2, ay / den2, az / den2

    s = jnp.sqrt(ax * ax + ay * ay)
    c = az
    near_zero = (s < 1e-8) & (c > 0)
    near_pi = (s < 1e-8) & (c < 0)
    use_e0 = jnp.abs(ax) < 0.9
    vax = jnp.where(use_e0, 0.0, -az)
    vay = jnp.where(use_e0, az, 0.0)
    vaz = jnp.where(use_e0, -ay, ax)
    dena = jnp.sqrt((vax * vax + vay * vay) + vaz * vaz) + 1e-8
    vax, vay, vaz = vax / dena, vay / dena, vaz / dena
    vx = jnp.where(near_pi, vax, ay)
    vy = jnp.where(near_pi, vay, -ax)
    vz = jnp.where(near_pi, vaz, 0.0)
    denk = jnp.sqrt((vx * vx + vy * vy) + vz * vz) + 1e-8
    kx, ky, kz = vx / denk, vy / denk, vz / denk

    zero = jnp.zeros_like(kx)
    K = [[zero, -kz, ky], [kz, zero, -kx], [-ky, kx, zero]]
    Kb = [[_b16(e) for e in row] for row in K]
    K2 = [[(Kb[i][0] * Kb[0][j] + Kb[i][1] * Kb[1][j]) + Kb[i][2] * Kb[2][j]
           for j in range(3)] for i in range(3)]
    t1 = 1.0 - c
    Rm = [[(1.0 if i == j else 0.0) + K[i][j] * s + K2[i][j] * t1
           for j in range(3)] for i in range(3)]
    Rm = [[jnp.where(near_zero, 1.0 if i == j else 0.0, Rm[i][j])
           for j in range(3)] for i in range(3)]

    xb, yb, zb = _b16(x), _b16(y), _b16(z)
    Rb = [[_b16(Rm[i][j]).reshape(TN, 1) for j in range(3)] for i in range(3)]
    xr = (xb * Rb[0][0] + yb * Rb[0][1]) + zb * Rb[0][2]
    yr = (xb * Rb[1][0] + yb * Rb[1][1]) + zb * Rb[1][2]
    zr = (xb * Rb[2][0] + yb * Rb[2][1]) + zb * Rb[2][2]

    rho = jnp.maximum(jnp.sqrt((xr * xr + yr * yr) + zr * zr), 1e-12)
    ct = zr / rho
    it_ = ((ct < _CT[0]).astype(jnp.int32) + (ct < _CT[1]).astype(jnp.int32)
           + (ct < _CT[2]).astype(jnp.int32) + (ct < _CT[3]).astype(jnp.int32)
           + (ct < _CT[4]).astype(jnp.int32))

    half = 0.5 * jnp.sqrt(xr * xr + yr * yr)
    ipos = (xr < half).astype(jnp.int32) + (xr < -half).astype(jnp.int32)
    ineg = 3 + (xr >= -half).astype(jnp.int32) + (xr >= half).astype(jnp.int32)
    ip_ = jnp.where(yr > 0, ipos,
                    jnp.where(yr < 0, ineg,
                              jnp.where(xr < 0, 2, 0)))

    rho_max = jnp.max(rho, axis=1, keepdims=True)
    hf = jnp.maximum(0.5 - rho / (2.0 * (rho_max + 1e-12)), 0.0) + 0.5
    return xr, yr, zr, hf, it_ * WBINS + ip_


def _feat_body(dp_ref, vals_ref, sidx_ref, gidx_ref):
    d = dp_ref[0]
    x, y, z = d[0], d[1], d[2]
    xr, yr, zr, hf, bins = _geom(x, y, z)
    vals_ref[...] = jnp.concatenate(
        [xr.T, yr.T, zr.T, hf.T], axis=0)
    colI = jax.lax.broadcasted_iota(jnp.int32, (x.shape[1], x.shape[0]), 1)
    base_t = bins.T * 4 + colI * HPAD
    sidx_ref[...] = jnp.concatenate(
        [base_t, base_t + 1, base_t + 2, base_t + 3], axis=0)
    rowi = jax.lax.broadcasted_iota(jnp.int32, bins.shape, 0)
    j = pl.program_id(1)
    gidx_ref[...] = ((j % 2) * 512 + rowi) * YPAD + bins


def _conv_body(hist_ref, mt_ref, b_ref, y_ref):
    y_ref[...] = (jnp.dot(hist_ref[...], mt_ref[...], preferred_element_type=F32)
                  + b_ref[0, 0])


def _make_sc_kernels(BN, G):
    mesh = plsc.VectorSubcoreMesh(core_axis_name="c", subcore_axis_name="s")
    A_TILE = BN // 32
    CHUNK = 512
    VPC = 4 * G
    SLABR = 32
    SLABW = SLABR * CHUNK

    @functools.partial(
        pl.kernel, mesh=mesh,
        compiler_params=pltpu.CompilerParams(needs_layout_passes=False),
        out_type=jax.ShapeDtypeStruct((BN * HPAD,), F32),
        scratch_types=[
            pltpu.VMEM((SLABR, CHUNK), F32),
            pltpu.VMEM((SLABR, CHUNK), jnp.int32),
            pltpu.VMEM((CHUNK * HPAD,), F32),
        ],
    )
    def scatter_k(vals_hbm, sidx_hbm, hist_hbm, vals_v, sidx_v, hist_v):
        wid = lax.axis_index("s") * 2 + lax.axis_index("c")
        zv = jnp.zeros((16,), F32)
        for chunk in range(A_TILE // CHUNK):
            cglob = wid * (A_TILE // CHUNK) + chunk
            abase = cglob * CHUNK

            def zbody(k, _):
                b0 = k * 128
                for u in range(8):
                    hist_v[pl.ds(b0 + u * 16, 16)] = zv
                return 0
            lax.fori_loop(0, CHUNK * HPAD // 128, zbody, 0)

            for slab in range(VPC // SLABR):
                rbase = cglob * VPC + slab * SLABR
                pltpu.sync_copy(vals_hbm.at[pl.ds(rbase, SLABR)], vals_v)
                pltpu.sync_copy(sidx_hbm.at[pl.ds(rbase, SLABR)], sidx_v)

                def sbody(i, _):
                    r = i >> 2
                    b0 = (i & 3) * 128
                    for u in range(8):
                        off = pl.ds(b0 + u * 16, 16)
                        plsc.addupdate_scatter(
                            hist_v, [sidx_v[r, off]], vals_v[r, off])
                    return 0
                lax.fori_loop(0, SLABR * (CHUNK // 128), sbody, 0)
            pltpu.sync_copy(hist_v, hist_hbm.at[pl.ds(abase * HPAD, CHUNK * HPAD)])

    P_TILE = A_TILE * G
    PSLAB = 8192

    ASLAB = PSLAB // G

    @functools.partial(
        pl.kernel, mesh=mesh,
        compiler_params=pltpu.CompilerParams(needs_layout_passes=False),
        out_type=jax.ShapeDtypeStruct((BN, G), F32),
        scratch_types=[
            pltpu.VMEM((A_TILE * YPAD,), F32),
            pltpu.VMEM((ASLAB, G), jnp.int32),
            pltpu.VMEM((ASLAB, G), F32),
        ],
    )
    def gather_k(y_hbm, gidx_hbm, out_hbm, y_v, gi_v, out_v):
        wid = lax.axis_index("s") * 2 + lax.axis_index("c")
        pltpu.sync_copy(y_hbm.at[pl.ds(wid * A_TILE * YPAD, A_TILE * YPAD)], y_v)
        for slab in range(P_TILE // PSLAB):
            abase = wid * A_TILE + slab * ASLAB
            pltpu.sync_copy(gidx_hbm.at[pl.ds(abase, ASLAB)], gi_v)

            def gbody(i, _):
                for u in range(G // 16):
                    off = pl.ds(u * 16, 16)
                    out_v[i, off] = plsc.load_gather(y_v, [gi_v[i, off]])
                return 0
            lax.fori_loop(0, ASLAB, gbody, 0)
            pltpu.sync_copy(out_v, out_hbm.at[pl.ds(abase, ASLAB)])

    return scatter_k, gather_k


@jax.jit
def kernel(dp, W, b):
    B, _, N, G = dp.shape
    BN = B * N
    TN = 512
    mt = _conv_matrix(W, b)

    nblk = (B * N) // TN
    vals, sidx, gidx = pl.pallas_call(
        _feat_body,
        grid=(B, N // TN),
        in_specs=[pl.BlockSpec((1, 3, TN, G), lambda i, j: (i, 0, j, 0))],
        out_specs=[
            pl.BlockSpec((4 * G, TN), lambda i, j: (i * (N // TN) + j, 0)),
            pl.BlockSpec((4 * G, TN), lambda i, j: (i * (N // TN) + j, 0)),
            pl.BlockSpec((TN, G), lambda i, j: (i * (N // TN) + j, 0)),
        ],
        out_shape=[
            jax.ShapeDtypeStruct((nblk * 4 * G, TN), F32),
            jax.ShapeDtypeStruct((nblk * 4 * G, TN), jnp.int32),
            jax.ShapeDtypeStruct((BN, G), jnp.int32),
        ],
    )(dp)

    scatter_k, gather_k = _make_sc_kernels(BN, G)
    hist = scatter_k(vals, sidx)

    TM = 2048
    y = pl.pallas_call(
        _conv_body,
        grid=(BN // TM,),
        in_specs=[
            pl.BlockSpec((TM, HPAD), lambda i: (i, 0)),
            pl.BlockSpec((HPAD, YPAD), lambda i: (0, 0)),
            pl.BlockSpec((1, 1), lambda i: (0, 0)),
        ],
        out_specs=pl.BlockSpec((TM, YPAD), lambda i: (i, 0)),
        out_shape=jax.ShapeDtypeStruct((BN, YPAD), F32),
    )(hist.reshape(BN, HPAD), mt, b.reshape(1, 1))

    out = gather_k(y.reshape(-1), gidx)
    return out.reshape(B, N, G)[:, None]


# speedup vs baseline: 25.5431x; 1.0792x over previous
import math
import functools

import numpy as np
import jax
import jax.numpy as jnp
from jax import lax
from jax.experimental import pallas as pl
from jax.experimental.pallas import tpu as pltpu, tpu_sc as plsc

HBINS, WBINS = 6, 6
KH, KW = 5, 5
NBINS = HBINS * WBINS
HPAD = NBINS * 4 + 1
YPAD = NBINS + 1
F32 = jnp.float32

_CT = [math.cos(j * math.pi / HBINS) for j in range(1, HBINS)]


def _conv_matrix(W, b):
    rows, cols, widx = [], [], []
    for oh in range(HBINS):
        for ow in range(WBINS):
            for kh in range(KH):
                th = oh + kh - (KH - 1) // 2
                if th < 0 or th >= HBINS:
                    continue
                for kw in range(KW):
                    pw = (ow + kw - (KW - 1) // 2) % WBINS
                    for c in range(4):
                        rows.append(oh * WBINS + ow)
                        cols.append((th * WBINS + pw) * 4 + c)
                        widx.append(c * KH * KW + kh * KW + kw)
    sel = np.zeros((HPAD * YPAD, KH * KW * 4), np.float32)
    for r, col, w in zip(rows, cols, widx):
        sel[col * YPAD + r, w] += 1.0
    return jnp.matmul(jnp.asarray(sel), W.reshape(-1)).reshape(HPAD, YPAD)


def _b16(v):
    u = jax.lax.bitcast_convert_type(v, jnp.uint32)
    r = (u + jnp.uint32(0x7FFF) + ((u >> 16) & jnp.uint32(1))) & jnp.uint32(0xFFFF0000)
    return jax.lax.bitcast_convert_type(r, F32)


def _geom(x, y, z):
    G = x.shape[1]
    rho_pre = jnp.sqrt((x * x + y * y) + z * z)
    rmax = jnp.max(rho_pre, axis=1, keepdims=True)
    iot = jax.lax.broadcasted_iota(jnp.int32, rho_pre.shape, 1)
    cand = jnp.where(rho_pre >= rmax, iot, G)
    fidx = jnp.min(cand, axis=1, keepdims=True)
    selm = (iot == fidx).astype(F32)
    dix = jnp.sum(x * selm, axis=1, keepdims=True)
    diy = jnp.sum(y * selm, axis=1, keepdims=True)
    diz = jnp.sum(z * selm, axis=1, keepdims=True)

    TN = x.shape[0]
    cshape = (TN // 128, 128)
    dix, diy, diz = (v.reshape(cshape) for v in (dix, diy, diz))

    den1 = jnp.sqrt((dix * dix + diy * diy) + diz * diz) + 1e-12
    ax, ay, az = dix / den1, diy / den1, diz / den1
    den2 = jnp.sqrt((ax * ax + ay * ay) + az * az) + 1e-8
    ax, ay, az = ax / den2, ay / den2, az / den2

    s = jnp.sqrt(ax * ax + ay * ay)
    c = az
    near_zero = (s < 1e-8) & (c > 0)
    near_pi = (s < 1e-8) & (c < 0)
    use_e0 = jnp.abs(ax) < 0.9
    vax = jnp.where(use_e0, 0.0, -az)
    vay = jnp.where(use_e0, az, 0.0)
    vaz = jnp.where(use_e0, -ay, ax)
    dena = jnp.sqrt((vax * vax + vay * vay) + vaz * vaz) + 1e-8
    vax, vay, vaz = vax / dena, vay / dena, vaz / dena
    vx = jnp.where(near_pi, vax, ay)
    vy = jnp.where(near_pi, vay, -ax)
    vz = jnp.where(near_pi, vaz, 0.0)
    denk = jnp.sqrt((vx * vx + vy * vy) + vz * vz) + 1e-8
    kx, ky, kz = vx / denk, vy / denk, vz / denk

    zero = jnp.zeros_like(kx)
    K = [[zero, -kz, ky], [kz, zero, -kx], [-ky, kx, zero]]
    Kb = [[_b16(e) for e in row] for row in K]
    K2 = [[(Kb[i][0] * Kb[0][j] + Kb[i][1] * Kb[1][j]) + Kb[i][2] * Kb[2][j]
           for j in range(3)] for i in range(3)]
    t1 = 1.0 - c
    Rm = [[(1.0 if i == j else 0.0) + K[i][j] * s + K2[i][j] * t1
           for j in range(3)] for i in range(3)]
    Rm = [[jnp.where(near_zero, 1.0 if i == j else 0.0, Rm[i][j])
           for j in range(3)] for i in range(3)]

    xb, yb, zb = _b16(x), _b16(y), _b16(z)
    Rb = [[_b16(Rm[i][j]).reshape(TN, 1) for j in range(3)] for i in range(3)]
    xr = (xb * Rb[0][0] + yb * Rb[0][1]) + zb * Rb[0][2]
    yr = (xb * Rb[1][0] + yb * Rb[1][1]) + zb * Rb[1][2]
    zr = (xb * Rb[2][0] + yb * Rb[2][1]) + zb * Rb[2][2]

    rho = jnp.maximum(jnp.sqrt((xr * xr + yr * yr) + zr * zr), 1e-12)
    ct = zr / rho
    it_ = ((ct < _CT[0]).astype(jnp.int32) + (ct < _CT[1]).astype(jnp.int32)
           + (ct < _CT[2]).astype(jnp.int32) + (ct < _CT[3]).astype(jnp.int32)
           + (ct < _CT[4]).astype(jnp.int32))

    half = 0.5 * jnp.sqrt(xr * xr + yr * yr)
    ipos = (xr < half).astype(jnp.int32) + (xr < -half).astype(jnp.int32)
    ineg = 3 + (xr >= -half).astype(jnp.int32) + (xr >= half).astype(jnp.int32)
    ip_ = jnp.where(yr > 0, ipos,
                    jnp.where(yr < 0, ineg,
                              jnp.where(xr < 0, 2, 0)))

    rho_max = jnp.max(rho, axis=1, keepdims=True)
    hf = jnp.maximum(0.5 - rho / (2.0 * (rho_max + 1e-12)), 0.0) + 0.5
    return xr, yr, zr, hf, it_ * WBINS + ip_


def _feat_body(dp_ref, vals_ref, sidx_ref, gidx_ref):
    d = dp_ref[0]
    x, y, z = d[0], d[1], d[2]
    xr, yr, zr, hf, bins = _geom(x, y, z)
    vals_ref[...] = jnp.concatenate(
        [xr.T, yr.T, zr.T, hf.T], axis=0)
    colI = jax.lax.broadcasted_iota(jnp.int32, (x.shape[1], x.shape[0]), 1)
    base_t = bins.T * 4 + colI * HPAD
    sidx_ref[...] = jnp.concatenate(
        [base_t, base_t + 1, base_t + 2, base_t + 3], axis=0)
    rowi = jax.lax.broadcasted_iota(jnp.int32, bins.shape, 0)
    j = pl.program_id(1)
    gidx_ref[...] = ((j % 2) * 512 + rowi) * YPAD + bins


def _conv_body(hist_ref, mt_ref, b_ref, y_ref):
    y_ref[...] = (jnp.dot(hist_ref[...], mt_ref[...], preferred_element_type=F32)
                  + b_ref[0, 0])


def _make_sc_kernels(BN, G):
    mesh = plsc.VectorSubcoreMesh(core_axis_name="c", subcore_axis_name="s")
    A_TILE = BN // 32
    CHUNK = 512
    VPC = 4 * G
    SLABR = 16
    NSLAB = VPC // SLABR

    @functools.partial(
        pl.kernel, mesh=mesh,
        compiler_params=pltpu.CompilerParams(needs_layout_passes=False),
        out_type=jax.ShapeDtypeStruct((BN * HPAD,), F32),
        scratch_types=[
            pltpu.VMEM((2, SLABR, CHUNK), F32),
            pltpu.VMEM((2, SLABR, CHUNK), jnp.int32),
            pltpu.VMEM((CHUNK * HPAD,), F32),
            pltpu.SemaphoreType.DMA,
            pltpu.SemaphoreType.DMA,
            pltpu.SemaphoreType.DMA,
            pltpu.SemaphoreType.DMA,
        ],
    )
    def scatter_k(vals_hbm, sidx_hbm, hist_hbm, vals_v, sidx_v, hist_v,
                  sv0, si0, sv1, si1):
        wid = lax.axis_index("s") * 2 + lax.axis_index("c")
        zv = jnp.zeros((16,), F32)
        svs = (sv0, sv1)
        sis = (si0, si1)

        def start(cglob, slab):
            rbase = cglob * VPC + slab * SLABR
            p = slab % 2
            hv = pltpu.async_copy(
                vals_hbm.at[pl.ds(rbase, SLABR)], vals_v.at[p], svs[p])
            hi = pltpu.async_copy(
                sidx_hbm.at[pl.ds(rbase, SLABR)], sidx_v.at[p], sis[p])
            return hv, hi

        for chunk in range(A_TILE // CHUNK):
            cglob = wid * (A_TILE // CHUNK) + chunk
            abase = cglob * CHUNK
            pend = start(cglob, 0)

            def zbody(k, _):
                b0 = k * 128
                for u in range(8):
                    hist_v[pl.ds(b0 + u * 16, 16)] = zv
                return 0
            lax.fori_loop(0, CHUNK * HPAD // 128, zbody, 0)

            for slab in range(NSLAB):
                p = slab % 2
                nxt = start(cglob, slab + 1) if slab + 1 < NSLAB else None
                pend[0].wait()
                pend[1].wait()
                pend = nxt

                def sbody(i, _):
                    r = i >> 2
                    b0 = (i & 3) * 128
                    for u in range(8):
                        off = pl.ds(b0 + u * 16, 16)
                        plsc.addupdate_scatter(
                            hist_v, [sidx_v[p, r, off]], vals_v[p, r, off])
                    return 0
                lax.fori_loop(0, SLABR * (CHUNK // 128), sbody, 0)
            pltpu.sync_copy(hist_v, hist_hbm.at[pl.ds(abase * HPAD, CHUNK * HPAD)])

    P_TILE = A_TILE * G
    PSLAB = 8192

    ASLAB = PSLAB // G

    @functools.partial(
        pl.kernel, mesh=mesh,
        compiler_params=pltpu.CompilerParams(needs_layout_passes=False),
        out_type=jax.ShapeDtypeStruct((BN, G), F32),
        scratch_types=[
            pltpu.VMEM((A_TILE * YPAD,), F32),
            pltpu.VMEM((ASLAB, G), jnp.int32),
            pltpu.VMEM((ASLAB, G), F32),
        ],
    )
    def gather_k(y_hbm, gidx_hbm, out_hbm, y_v, gi_v, out_v):
        wid = lax.axis_index("s") * 2 + lax.axis_index("c")
        pltpu.sync_copy(y_hbm.at[pl.ds(wid * A_TILE * YPAD, A_TILE * YPAD)], y_v)
        for slab in range(P_TILE // PSLAB):
            abase = wid * A_TILE + slab * ASLAB
            pltpu.sync_copy(gidx_hbm.at[pl.ds(abase, ASLAB)], gi_v)

            def gbody(i, _):
                for u in range(G // 16):
                    off = pl.ds(u * 16, 16)
                    out_v[i, off] = plsc.load_gather(y_v, [gi_v[i, off]])
                return 0
            lax.fori_loop(0, ASLAB, gbody, 0)
            pltpu.sync_copy(out_v, out_hbm.at[pl.ds(abase, ASLAB)])

    return scatter_k, gather_k


@jax.jit
def kernel(dp, W, b):
    B, _, N, G = dp.shape
    BN = B * N
    TN = 512
    mt = _conv_matrix(W, b)

    nblk = (B * N) // TN
    vals, sidx, gidx = pl.pallas_call(
        _feat_body,
        grid=(B, N // TN),
        in_specs=[pl.BlockSpec((1, 3, TN, G), lambda i, j: (i, 0, j, 0))],
        out_specs=[
            pl.BlockSpec((4 * G, TN), lambda i, j: (i * (N // TN) + j, 0)),
            pl.BlockSpec((4 * G, TN), lambda i, j: (i * (N // TN) + j, 0)),
            pl.BlockSpec((TN, G), lambda i, j: (i * (N // TN) + j, 0)),
        ],
        out_shape=[
            jax.ShapeDtypeStruct((nblk * 4 * G, TN), F32),
            jax.ShapeDtypeStruct((nblk * 4 * G, TN), jnp.int32),
            jax.ShapeDtypeStruct((BN, G), jnp.int32),
        ],
    )(dp)

    scatter_k, gather_k = _make_sc_kernels(BN, G)
    hist = scatter_k(vals, sidx)

    TM = 2048
    y = pl.pallas_call(
        _conv_body,
        grid=(BN // TM,),
        in_specs=[
            pl.BlockSpec((TM, HPAD), lambda i: (i, 0)),
            pl.BlockSpec((HPAD, YPAD), lambda i: (0, 0)),
            pl.BlockSpec((1, 1), lambda i: (0, 0)),
        ],
        out_specs=pl.BlockSpec((TM, YPAD), lambda i: (i, 0)),
        out_shape=jax.ShapeDtypeStruct((BN, YPAD), F32),
    )(hist.reshape(BN, HPAD), mt, b.reshape(1, 1))

    out = gather_k(y.reshape(-1), gidx)
    return out.reshape(B, N, G)[:, None]


# speedup vs baseline: 34.5743x; 1.3536x over previous
import math
import functools

import numpy as np
import jax
import jax.numpy as jnp
from jax import lax
from jax.experimental import pallas as pl
from jax.experimental.pallas import tpu as pltpu, tpu_sc as plsc

HBINS, WBINS = 6, 6
KH, KW = 5, 5
NBINS = HBINS * WBINS
HPAD = NBINS * 4 + 1
YPAD = NBINS + 1
F32 = jnp.float32

_CT = [math.cos(j * math.pi / HBINS) for j in range(1, HBINS)]


def _conv_matrix(W, b):
    rows, cols, widx = [], [], []
    for oh in range(HBINS):
        for ow in range(WBINS):
            for kh in range(KH):
                th = oh + kh - (KH - 1) // 2
                if th < 0 or th >= HBINS:
                    continue
                for kw in range(KW):
                    pw = (ow + kw - (KW - 1) // 2) % WBINS
                    for c in range(4):
                        rows.append(oh * WBINS + ow)
                        cols.append((th * WBINS + pw) * 4 + c)
                        widx.append(c * KH * KW + kh * KW + kw)
    sel = np.zeros((HPAD * YPAD, KH * KW * 4), np.float32)
    for r, col, w in zip(rows, cols, widx):
        sel[col * YPAD + r, w] += 1.0
    return jnp.matmul(jnp.asarray(sel), W.reshape(-1)).reshape(HPAD, YPAD)


def _b16(v):
    u = jax.lax.bitcast_convert_type(v, jnp.uint32)
    r = (u + jnp.uint32(0x7FFF) + ((u >> 16) & jnp.uint32(1))) & jnp.uint32(0xFFFF0000)
    return jax.lax.bitcast_convert_type(r, F32)


def _geom(x, y, z):
    G = x.shape[0]
    rho_pre = jnp.sqrt((x * x + y * y) + z * z)
    rmax = jnp.max(rho_pre, axis=0, keepdims=True)
    iot = jax.lax.broadcasted_iota(jnp.int32, rho_pre.shape, 0)
    cand = jnp.where(rho_pre >= rmax, iot, G)
    fidx = jnp.min(cand, axis=0, keepdims=True)
    selm = (iot == fidx).astype(F32)
    dix = jnp.sum(x * selm, axis=0, keepdims=True)
    diy = jnp.sum(y * selm, axis=0, keepdims=True)
    diz = jnp.sum(z * selm, axis=0, keepdims=True)

    den1 = jnp.sqrt((dix * dix + diy * diy) + diz * diz) + 1e-12
    ax, ay, az = dix / den1, diy / den1, diz / den1
    den2 = jnp.sqrt((ax * ax + ay * ay) + az * az) + 1e-8
    ax, ay, az = ax / den2, ay / den2, az / den2

    s = jnp.sqrt(ax * ax + ay * ay)
    c = az
    near_zero = (s < 1e-8) & (c > 0)
    near_pi = (s < 1e-8) & (c < 0)
    use_e0 = jnp.abs(ax) < 0.9
    vax = jnp.where(use_e0, 0.0, -az)
    vay = jnp.where(use_e0, az, 0.0)
    vaz = jnp.where(use_e0, -ay, ax)
    dena = jnp.sqrt((vax * vax + vay * vay) + vaz * vaz) + 1e-8
    vax, vay, vaz = vax / dena, vay / dena, vaz / dena
    vx = jnp.where(near_pi, vax, ay)
    vy = jnp.where(near_pi, vay, -ax)
    vz = jnp.where(near_pi, vaz, 0.0)
    denk = jnp.sqrt((vx * vx + vy * vy) + vz * vz) + 1e-8
    kx, ky, kz = vx / denk, vy / denk, vz / denk

    zero = jnp.zeros_like(kx)
    K = [[zero, -kz, ky], [kz, zero, -kx], [-ky, kx, zero]]
    Kb = [[_b16(e) for e in row] for row in K]
    K2 = [[(Kb[i][0] * Kb[0][j] + Kb[i][1] * Kb[1][j]) + Kb[i][2] * Kb[2][j]
           for j in range(3)] for i in range(3)]
    t1 = 1.0 - c
    Rm = [[(1.0 if i == j else 0.0) + K[i][j] * s + K2[i][j] * t1
           for j in range(3)] for i in range(3)]
    Rm = [[jnp.where(near_zero, 1.0 if i == j else 0.0, Rm[i][j])
           for j in range(3)] for i in range(3)]

    xb, yb, zb = _b16(x), _b16(y), _b16(z)
    Rb = [[_b16(Rm[i][j]) for j in range(3)] for i in range(3)]
    xr = (xb * Rb[0][0] + yb * Rb[0][1]) + zb * Rb[0][2]
    yr = (xb * Rb[1][0] + yb * Rb[1][1]) + zb * Rb[1][2]
    zr = (xb * Rb[2][0] + yb * Rb[2][1]) + zb * Rb[2][2]

    rho = jnp.maximum(jnp.sqrt((xr * xr + yr * yr) + zr * zr), 1e-12)
    ct = zr / rho
    it_ = ((ct < _CT[0]).astype(jnp.int32) + (ct < _CT[1]).astype(jnp.int32)
           + (ct < _CT[2]).astype(jnp.int32) + (ct < _CT[3]).astype(jnp.int32)
           + (ct < _CT[4]).astype(jnp.int32))

    half = 0.5 * jnp.sqrt(xr * xr + yr * yr)
    ipos = (xr < half).astype(jnp.int32) + (xr < -half).astype(jnp.int32)
    ineg = 3 + (xr >= -half).astype(jnp.int32) + (xr >= half).astype(jnp.int32)
    ip_ = jnp.where(yr > 0, ipos,
                    jnp.where(yr < 0, ineg,
                              jnp.where(xr < 0, 2, 0)))

    rho_max = jnp.max(rho, axis=0, keepdims=True)
    hf = jnp.maximum(0.5 - rho / (2.0 * (rho_max + 1e-12)), 0.0) + 0.5
    return xr, yr, zr, hf, it_ * WBINS + ip_


def _feat_body(dp_ref, vals_ref, sidx_ref, gidx_ref):
    d = dp_ref[0]
    x, y, z = d[0].T, d[1].T, d[2].T
    xr, yr, zr, hf, bins_t = _geom(x, y, z)
    vals_ref[...] = jnp.concatenate(
        [xr, yr, zr, hf], axis=0)
    colI = jax.lax.broadcasted_iota(jnp.int32, bins_t.shape, 1)
    base_t = bins_t * 4 + colI * HPAD
    sidx_ref[...] = jnp.concatenate(
        [base_t, base_t + 1, base_t + 2, base_t + 3], axis=0)
    rowi = jax.lax.broadcasted_iota(
        jnp.int32, (bins_t.shape[1], bins_t.shape[0]), 0)
    j = pl.program_id(1)
    gidx_ref[...] = ((j % 2) * 512 + rowi) * YPAD + bins_t.T


def _conv_body(hist_ref, mt_ref, b_ref, y_ref):
    y_ref[...] = (jnp.dot(hist_ref[...], mt_ref[...], preferred_element_type=F32)
                  + b_ref[0, 0])


def _make_sc_kernels(BN, G):
    mesh = plsc.VectorSubcoreMesh(core_axis_name="c", subcore_axis_name="s")
    A_TILE = BN // 32
    CHUNK = 512
    VPC = 4 * G
    SLABR = 16
    NSLAB = VPC // SLABR

    @functools.partial(
        pl.kernel, mesh=mesh,
        compiler_params=pltpu.CompilerParams(needs_layout_passes=False),
        out_type=jax.ShapeDtypeStruct((BN * HPAD,), F32),
        scratch_types=[
            pltpu.VMEM((2, SLABR, CHUNK), F32),
            pltpu.VMEM((2, SLABR, CHUNK), jnp.int32),
            pltpu.VMEM((CHUNK * HPAD,), F32),
            pltpu.SemaphoreType.DMA,
            pltpu.SemaphoreType.DMA,
            pltpu.SemaphoreType.DMA,
            pltpu.SemaphoreType.DMA,
        ],
    )
    def scatter_k(vals_hbm, sidx_hbm, hist_hbm, vals_v, sidx_v, hist_v,
                  sv0, si0, sv1, si1):
        wid = lax.axis_index("s") * 2 + lax.axis_index("c")
        zv = jnp.zeros((16,), F32)
        svs = (sv0, sv1)
        sis = (si0, si1)

        def start(cglob, slab):
            rbase = cglob * VPC + slab * SLABR
            p = slab % 2
            hv = pltpu.async_copy(
                vals_hbm.at[pl.ds(rbase, SLABR)], vals_v.at[p], svs[p])
            hi = pltpu.async_copy(
                sidx_hbm.at[pl.ds(rbase, SLABR)], sidx_v.at[p], sis[p])
            return hv, hi

        for chunk in range(A_TILE // CHUNK):
            cglob = wid * (A_TILE // CHUNK) + chunk
            abase = cglob * CHUNK
            pend = start(cglob, 0)

            def zbody(k, _):
                b0 = k * 128
                for u in range(8):
                    hist_v[pl.ds(b0 + u * 16, 16)] = zv
                return 0
            lax.fori_loop(0, CHUNK * HPAD // 128, zbody, 0)

            for slab in range(NSLAB):
                p = slab % 2
                nxt = start(cglob, slab + 1) if slab + 1 < NSLAB else None
                pend[0].wait()
                pend[1].wait()
                pend = nxt

                def sbody(i, _):
                    r = i >> 2
                    b0 = (i & 3) * 128
                    for u in range(8):
                        off = pl.ds(b0 + u * 16, 16)
                        plsc.addupdate_scatter(
                            hist_v, [sidx_v[p, r, off]], vals_v[p, r, off])
                    return 0
                lax.fori_loop(0, SLABR * (CHUNK // 128), sbody, 0)
            pltpu.sync_copy(hist_v, hist_hbm.at[pl.ds(abase * HPAD, CHUNK * HPAD)])

    P_TILE = A_TILE * G
    PSLAB = 8192

    ASLAB = PSLAB // G

    @functools.partial(
        pl.kernel, mesh=mesh,
        compiler_params=pltpu.CompilerParams(needs_layout_passes=False),
        out_type=jax.ShapeDtypeStruct((BN, G), F32),
        scratch_types=[
            pltpu.VMEM((A_TILE * YPAD,), F32),
            pltpu.VMEM((ASLAB, G), jnp.int32),
            pltpu.VMEM((ASLAB, G), F32),
        ],
    )
    def gather_k(y_hbm, gidx_hbm, out_hbm, y_v, gi_v, out_v):
        wid = lax.axis_index("s") * 2 + lax.axis_index("c")
        pltpu.sync_copy(y_hbm.at[pl.ds(wid * A_TILE * YPAD, A_TILE * YPAD)], y_v)
        for slab in range(P_TILE // PSLAB):
            abase = wid * A_TILE + slab * ASLAB
            pltpu.sync_copy(gidx_hbm.at[pl.ds(abase, ASLAB)], gi_v)

            def gbody(i, _):
                for u in range(G // 16):
                    off = pl.ds(u * 16, 16)
                    out_v[i, off] = plsc.load_gather(y_v, [gi_v[i, off]])
                return 0
            lax.fori_loop(0, ASLAB, gbody, 0)
            pltpu.sync_copy(out_v, out_hbm.at[pl.ds(abase, ASLAB)])

    return scatter_k, gather_k


@jax.jit
def kernel(dp, W, b):
    B, _, N, G = dp.shape
    BN = B * N
    TN = 512
    mt = _conv_matrix(W, b)

    nblk = (B * N) // TN
    vals, sidx, gidx = pl.pallas_call(
        _feat_body,
        grid=(B, N // TN),
        in_specs=[pl.BlockSpec((1, 3, TN, G), lambda i, j: (i, 0, j, 0))],
        out_specs=[
            pl.BlockSpec((4 * G, TN), lambda i, j: (i * (N // TN) + j, 0)),
            pl.BlockSpec((4 * G, TN), lambda i, j: (i * (N // TN) + j, 0)),
            pl.BlockSpec((TN, G), lambda i, j: (i * (N // TN) + j, 0)),
        ],
        out_shape=[
            jax.ShapeDtypeStruct((nblk * 4 * G, TN), F32),
            jax.ShapeDtypeStruct((nblk * 4 * G, TN), jnp.int32),
            jax.ShapeDtypeStruct((BN, G), jnp.int32),
        ],
    )(dp)

    scatter_k, gather_k = _make_sc_kernels(BN, G)
    hist = scatter_k(vals, sidx)

    TM = 2048
    y = pl.pallas_call(
        _conv_body,
        grid=(BN // TM,),
        in_specs=[
            pl.BlockSpec((TM, HPAD), lambda i: (i, 0)),
            pl.BlockSpec((HPAD, YPAD), lambda i: (0, 0)),
            pl.BlockSpec((1, 1), lambda i: (0, 0)),
        ],
        out_specs=pl.BlockSpec((TM, YPAD), lambda i: (i, 0)),
        out_shape=jax.ShapeDtypeStruct((BN, YPAD), F32),
    )(hist.reshape(BN, HPAD), mt, b.reshape(1, 1))

    out = gather_k(y.reshape(-1), gidx)
    return out.reshape(B, N, G)[:, None]
